# baseline (speedup 1.0000x reference)
import functools

import jax
import jax.numpy as jnp
from jax import lax
from jax.experimental import pallas as pl
from jax.experimental.pallas import tpu as pltpu
from jax.experimental.pallas import tpu_sc as plsc

LANES = 16


def _build_obs_table(emb_obs, b_val):
    max_n, d = emb_obs.shape

    def body(eo_ref, b_ref, out_ref):
        out_ref[...] = eo_ref[...] + b_ref[...][None, :]

    return pl.pallas_call(
        body,
        out_shape=jax.ShapeDtypeStruct((max_n, d), jnp.float32),
    )(emb_obs, b_val)


@jax.jit
def _sc_embed5(tbl_obs, tbl_feat, w, obs_t, feat_t, val_t):
    l_dim, b_dim = obs_t.shape
    max_n, d = tbl_obs.shape
    di = tbl_feat.shape[0]
    info = plsc.get_sparse_core_info()
    nw = info.num_cores * info.num_subcores
    n_dblk, n_dsub = d // 8, 8
    n_bblk, n_bsub = b_dim // 128, 128
    ub = 512
    uq = b_dim // ub
    n_units = l_dim * uq
    per_w = n_units // nw
    assert n_units % nw == 0 and per_w % 2 == 0
    n_grp = ub // LANES
    ubblk = ub // 128
    mesh = plsc.VectorSubcoreMesh(core_axis_name="c", subcore_axis_name="s")

    @functools.partial(
        pl.kernel,
        out_type=jax.ShapeDtypeStruct((l_dim, n_dblk, n_bblk, n_dsub, n_bsub),
                                      jnp.float32),
        mesh=mesh,
        compiler_params=pltpu.CompilerParams(
            use_tc_tiling_on_sc=False, needs_layout_passes=False),
        scratch_types=[
            pltpu.VMEM((max_n, d), jnp.float32),
            pltpu.VMEM((di, d), jnp.float32),
            pltpu.VMEM((d,), jnp.float32),
            pltpu.VMEM((ub,), jnp.int32),
            pltpu.VMEM((ub,), jnp.int32),
            pltpu.VMEM((ub,), jnp.int32),
            pltpu.VMEM((ub,), jnp.int32),
            pltpu.VMEM((ub,), jnp.float32),
            pltpu.VMEM((ub,), jnp.float32),
            pltpu.VMEM((n_dblk, ubblk, n_dsub, n_bsub), jnp.float32),
            pltpu.VMEM((n_dblk, ubblk, n_dsub, n_bsub), jnp.float32),
            pltpu.SemaphoreType.DMA,
            pltpu.SemaphoreType.DMA,
            pltpu.SemaphoreType.DMA,
            pltpu.SemaphoreType.DMA,
        ],
    )
    def k(tobs_hbm, tfeat_hbm, w_hbm, obs_hbm, feat_hbm, val_hbm, out_hbm,
          ta_v, tf_v, w_v, obs0, obs1, feat0, feat1, val0, val1,
          ot0, ot1, si0, si1, so0, so1):
        OBS, FEAT, VAL = [obs0, obs1], [feat0, feat1], [val0, val1]
        OT, SI, SO = [ot0, ot1], [si0, si1], [so0, so1]
        wid = lax.axis_index("s") * info.num_cores + lax.axis_index("c")
        pltpu.sync_copy(tobs_hbm, ta_v)
        pltpu.sync_copy(tfeat_hbm, tf_v)
        pltpu.sync_copy(w_hbm, w_v)
        wregs = [w_v[pl.ds(t * LANES, LANES)] for t in range(d // LANES)]

        def unit_lq(u):
            gu = wid * per_w + u
            return gu // uq, gu % uq

        def issue_in(u, p):
            l, q = unit_lq(lax.min(u, per_w - 1))
            sl = pl.ds(q * ub, ub)
            pltpu.async_copy(obs_hbm.at[l, sl], OBS[p], SI[p])
            pltpu.async_copy(feat_hbm.at[l, sl], FEAT[p], SI[p])
            pltpu.async_copy(val_hbm.at[l, sl], VAL[p], SI[p])

        def wait_in(p):
            sl = pl.ds(0, ub)
            pltpu.make_async_copy(obs_hbm.at[0, sl], OBS[p], SI[p]).wait()
            pltpu.make_async_copy(feat_hbm.at[0, sl], FEAT[p], SI[p]).wait()
            pltpu.make_async_copy(val_hbm.at[0, sl], VAL[p], SI[p]).wait()

        def issue_out(u, p):
            l, q = unit_lq(u)
            for dblk in range(n_dblk):
                pltpu.async_copy(OT[p].at[dblk],
                                 out_hbm.at[l, dblk, pl.ds(q * ubblk, ubblk)],
                                 SO[p])

        def wait_out(p):
            for dblk in range(n_dblk):
                pltpu.make_async_copy(
                    OT[p].at[dblk],
                    out_hbm.at[0, dblk, pl.ds(0, ubblk)], SO[p]).wait()

        def compute(p):
            ot = OT[p]

            def grp_body(j, _):
                obs16 = OBS[p][pl.ds(j * LANES, LANES)]
                feat16 = FEAT[p][pl.ds(j * LANES, LANES)]
                val16 = VAL[p][pl.ds(j * LANES, LANES)]
                jj = j // 8
                cc = (j % 8) * LANES
                for d_i in range(d):
                    wbc = lax.gather(
                        wregs[d_i // LANES],
                        jnp.full((LANES, 1), d_i % LANES, jnp.int32),
                        dimension_numbers=lax.GatherDimensionNumbers(
                            offset_dims=(), collapsed_slice_dims=(0,),
                            start_index_map=(0,)),
                        slice_sizes=(1,),
                        mode=lax.GatherScatterMode.PROMISE_IN_BOUNDS)
                    dcol = jnp.full((LANES,), d_i, jnp.int32)
                    go = plsc.load_gather(ta_v, [obs16, dcol])
                    gf = plsc.load_gather(tf_v, [feat16, dcol])
                    acc = go + gf + val16 * wbc
                    ot[d_i // 8, jj, d_i % 8, pl.ds(cc, LANES)] = acc
                return 0

            lax.fori_loop(0, n_grp, grp_body, 0)

        issue_in(0, 0)
        issue_in(1, 1)
        wait_in(0)
        compute(0)
        issue_out(0, 0)
        issue_in(2, 0)
        wait_in(1)
        compute(1)
        issue_out(1, 1)
        issue_in(3, 1)

        def pair_body(i, _):
            u0 = 2 * i
            wait_in(0)
            wait_out(0)
            compute(0)
            issue_out(u0, 0)
            issue_in(u0 + 2, 0)
            wait_in(1)
            wait_out(1)
            compute(1)
            issue_out(u0 + 1, 1)
            issue_in(u0 + 3, 1)
            return 0

        lax.fori_loop(1, per_w // 2, pair_body, 0)

        wait_in(0)
        wait_in(1)
        wait_out(0)
        wait_out(1)

    return k(tbl_obs, tbl_feat, w, obs_t, feat_t, val_t)


def kernel(val, obs_idx, feat_idx, W_val, b_val, emb_obs, emb_feat):
    b, l, _ = val.shape
    d = emb_obs.shape[1]
    tbl_obs = _build_obs_table(emb_obs, b_val)
    out5 = _sc_embed5(
        tbl_obs,
        emb_feat,
        W_val.reshape(d),
        obs_idx.astype(jnp.int32).T,
        feat_idx.astype(jnp.int32).T,
        val.reshape(b, l).T,
    )
    return out5.transpose(2, 4, 0, 1, 3).reshape(b, l, d)


# speedup vs baseline: 2.8570x; 2.8570x over previous
import functools

import jax
import jax.numpy as jnp
from jax import lax
from jax.experimental import pallas as pl
from jax.experimental.pallas import tpu as pltpu
from jax.experimental.pallas import tpu_sc as plsc

LANES = 16
TW = 72
GATHER_SUB = 128


def _build_table(emb_obs, emb_feat, b_val):
    max_n, d = emb_obs.shape
    di = emb_feat.shape[0]

    def body(eo_ref, ef_ref, b_ref, out_ref):
        tbl = eo_ref[...][:, None, :] + ef_ref[...][None, :, :] + b_ref[...][None, None, :]
        out_ref[:, 0:d] = tbl.reshape(max_n * di, d)
        out_ref[:, d:TW] = jnp.zeros((max_n * di, TW - d), jnp.float32)

    return pl.pallas_call(
        body,
        out_shape=jax.ShapeDtypeStruct((max_n * di, TW), jnp.float32),
    )(emb_obs, emb_feat, b_val)


@functools.partial(jax.jit, static_argnames=("di",))
def _sc_embed5(tbl, w, obs_t, feat_t, val_t, *, di):
    l_dim, b_dim = obs_t.shape
    d = 64
    info = plsc.get_sparse_core_info()
    nw = info.num_cores * info.num_subcores
    n_dblk, n_dsub = d // 8, 8
    n_bblk, n_bsub = b_dim // 128, 128
    ub = 256
    uq = b_dim // ub
    n_units = l_dim * uq
    per_w = n_units // nw
    assert n_units % nw == 0 and per_w % 2 == 0
    n_grp = ub // LANES
    n_sub = ub // GATHER_SUB
    ubblk = ub // 128
    mesh = plsc.VectorSubcoreMesh(core_axis_name="c", subcore_axis_name="s")

    @functools.partial(
        pl.kernel,
        out_type=jax.ShapeDtypeStruct((l_dim, n_dblk, n_bblk, n_dsub, n_bsub),
                                      jnp.float32),
        mesh=mesh,
        compiler_params=pltpu.CompilerParams(
            use_tc_tiling_on_sc=False, needs_layout_passes=False),
        scratch_types=[
            pltpu.VMEM((d,), jnp.float32),
            pltpu.VMEM((ub,), jnp.int32),
            pltpu.VMEM((ub,), jnp.int32),
            pltpu.VMEM((ub,), jnp.int32),
            pltpu.VMEM((ub,), jnp.int32),
            pltpu.VMEM((ub,), jnp.float32),
            pltpu.VMEM((ub,), jnp.float32),
            pltpu.VMEM((ub,), jnp.int32),
            pltpu.VMEM((ub,), jnp.int32),
            pltpu.VMEM((ub, TW), jnp.float32),
            pltpu.VMEM((ub, TW), jnp.float32),
            pltpu.VMEM((n_dblk, ubblk, n_dsub, n_bsub), jnp.float32),
            pltpu.VMEM((n_dblk, ubblk, n_dsub, n_bsub), jnp.float32),
            pltpu.SemaphoreType.DMA,
            pltpu.SemaphoreType.DMA,
            pltpu.SemaphoreType.DMA,
            pltpu.SemaphoreType.DMA,
            pltpu.SemaphoreType.DMA,
            pltpu.SemaphoreType.DMA,
        ],
    )
    def k(tbl_hbm, w_hbm, obs_hbm, feat_hbm, val_hbm, out_hbm,
          w_v, obs0, obs1, feat0, feat1, val0, val1, idx0, idx1,
          s0, s1, ot0, ot1, si0, si1, sg0, sg1, so0, so1):
        OBS, FEAT, VAL = [obs0, obs1], [feat0, feat1], [val0, val1]
        IDX, S, OT = [idx0, idx1], [s0, s1], [ot0, ot1]
        SI, SG, SO = [si0, si1], [sg0, sg1], [so0, so1]
        wid = lax.axis_index("s") * info.num_cores + lax.axis_index("c")
        pltpu.sync_copy(w_hbm, w_v)
        wregs = [w_v[pl.ds(t * LANES, LANES)] for t in range(d // LANES)]

        def unit_lq(u):
            gu = wid * per_w + u
            return gu // uq, gu % uq

        def issue_in(u, p):
            l, q = unit_lq(lax.min(u, per_w - 1))
            sl = pl.ds(q * ub, ub)
            pltpu.async_copy(obs_hbm.at[l, sl], OBS[p], SI[p])
            pltpu.async_copy(feat_hbm.at[l, sl], FEAT[p], SI[p])
            pltpu.async_copy(val_hbm.at[l, sl], VAL[p], SI[p])

        def wait_in(p):
            sl = pl.ds(0, ub)
            pltpu.make_async_copy(obs_hbm.at[0, sl], OBS[p], SI[p]).wait()
            pltpu.make_async_copy(feat_hbm.at[0, sl], FEAT[p], SI[p]).wait()
            pltpu.make_async_copy(val_hbm.at[0, sl], VAL[p], SI[p]).wait()

        def combine(p):
            for i in range(n_grp):
                sl = pl.ds(i * LANES, LANES)
                IDX[p][sl] = OBS[p][sl] * di + FEAT[p][sl]

        def issue_gather(p):
            for j in range(n_sub):
                sl = pl.ds(j * GATHER_SUB, GATHER_SUB)
                pltpu.async_copy(tbl_hbm.at[IDX[p].at[sl]], S[p].at[sl], SG[p])

        def wait_gather(p):
            for j in range(n_sub):
                sl = pl.ds(j * GATHER_SUB, GATHER_SUB)
                pltpu.make_async_copy(tbl_hbm.at[IDX[p].at[sl]], S[p].at[sl],
                                      SG[p]).wait()

        def issue_out(u, p):
            l, q = unit_lq(u)
            for dblk in range(n_dblk):
                pltpu.async_copy(OT[p].at[dblk],
                                 out_hbm.at[l, dblk, pl.ds(q * ubblk, ubblk)],
                                 SO[p])

        def wait_out(p):
            for dblk in range(n_dblk):
                pltpu.make_async_copy(
                    OT[p].at[dblk],
                    out_hbm.at[0, dblk, pl.ds(0, ubblk)], SO[p]).wait()

        def compute(p):
            s_v, ot = S[p], OT[p]

            def grp_body(j, _):
                val16 = VAL[p][pl.ds(j * LANES, LANES)]
                riota = lax.iota(jnp.int32, LANES) + j * LANES
                jj = j // 8
                cc = (j % 8) * LANES
                for dg in range(d // LANES):
                    wreg = wregs[dg]
                    for kk in range(LANES):
                        d_i = dg * LANES + kk
                        wbc = lax.gather(
                            wreg, jnp.full((LANES, 1), kk, jnp.int32),
                            dimension_numbers=lax.GatherDimensionNumbers(
                                offset_dims=(), collapsed_slice_dims=(0,),
                                start_index_map=(0,)),
                            slice_sizes=(1,),
                            mode=lax.GatherScatterMode.PROMISE_IN_BOUNDS)
                        col = plsc.load_gather(
                            s_v, [riota, jnp.full((LANES,), d_i, jnp.int32)])
                        ot[d_i // 8, jj, d_i % 8, pl.ds(cc, LANES)] = (
                            col + val16 * wbc)
                return 0

            lax.fori_loop(0, n_grp, grp_body, 0)

        issue_in(0, 0)
        issue_in(1, 1)
        wait_in(0)
        combine(0)
        issue_gather(0)
        wait_in(1)
        combine(1)
        issue_gather(1)
        wait_gather(0)
        compute(0)
        issue_out(0, 0)
        issue_in(2, 0)
        wait_in(0)
        combine(0)
        issue_gather(0)
        wait_gather(1)
        compute(1)
        issue_out(1, 1)
        issue_in(3, 1)

        def pair_body(i, _):
            u0 = 2 * i
            wait_in(1)
            combine(1)
            issue_gather(1)
            wait_gather(0)
            wait_out(0)
            compute(0)
            issue_out(u0, 0)
            issue_in(u0 + 2, 0)
            wait_in(0)
            combine(0)
            issue_gather(0)
            wait_gather(1)
            wait_out(1)
            compute(1)
            issue_out(u0 + 1, 1)
            issue_in(u0 + 3, 1)
            return 0

        lax.fori_loop(1, per_w // 2, pair_body, 0)

        wait_gather(0)
        wait_in(1)
        wait_out(0)
        wait_out(1)

    return k(tbl, w, obs_t, feat_t, val_t)


def kernel(val, obs_idx, feat_idx, W_val, b_val, emb_obs, emb_feat):
    b, l, _ = val.shape
    d = emb_obs.shape[1]
    di = emb_feat.shape[0]
    tbl = _build_table(emb_obs, emb_feat, b_val)
    out5 = _sc_embed5(
        tbl,
        W_val.reshape(d),
        obs_idx.astype(jnp.int32).T,
        feat_idx.astype(jnp.int32).T,
        val.reshape(b, l).T,
        di=di,
    )
    return out5.transpose(2, 4, 0, 1, 3).reshape(b, l, d)


# speedup vs baseline: 4.2119x; 1.4742x over previous
import functools

import jax
import jax.numpy as jnp
from jax import lax
from jax.experimental import pallas as pl
from jax.experimental.pallas import tpu as pltpu
from jax.experimental.pallas import tpu_sc as plsc

LANES = 16
TW = 72
GATHER_SUB = 128


def _build_table(emb_obs, emb_feat, b_val):
    max_n, d = emb_obs.shape
    di = emb_feat.shape[0]

    def body(eo_ref, ef_ref, b_ref, out_ref):
        tbl = eo_ref[...][:, None, :] + ef_ref[...][None, :, :] + b_ref[...][None, None, :]
        out_ref[:, 0:d] = tbl.reshape(max_n * di, d)
        out_ref[:, d:TW] = jnp.zeros((max_n * di, TW - d), jnp.float32)

    return pl.pallas_call(
        body,
        out_shape=jax.ShapeDtypeStruct((max_n * di, TW), jnp.float32),
    )(emb_obs, emb_feat, b_val)


@functools.partial(jax.jit, static_argnames=("di",))
def _sc_embed5(tbl, w, obs_t, feat_t, val_t, *, di):
    l_dim, b_dim = obs_t.shape
    d = 64
    info = plsc.get_sparse_core_info()
    nw = info.num_cores * info.num_subcores
    n_dblk, n_dsub = d // 8, 8
    n_bblk, n_bsub = b_dim // 128, 128
    ub = 256
    uq = b_dim // ub
    n_units = l_dim * uq
    per_w = n_units // nw
    assert n_units % nw == 0 and per_w % 2 == 0
    n_grp = ub // LANES
    n_sub = ub // GATHER_SUB
    ubblk = ub // 128
    mesh = plsc.VectorSubcoreMesh(core_axis_name="c", subcore_axis_name="s")

    @functools.partial(
        pl.kernel,
        out_type=jax.ShapeDtypeStruct((l_dim, n_dblk, n_bblk, n_dsub, n_bsub),
                                      jnp.float32),
        mesh=mesh,
        compiler_params=pltpu.CompilerParams(
            use_tc_tiling_on_sc=False, needs_layout_passes=False),
        scratch_types=[
            pltpu.VMEM((d,), jnp.float32),
            pltpu.VMEM((ub,), jnp.int32),
            pltpu.VMEM((ub,), jnp.int32),
            pltpu.VMEM((ub,), jnp.int32),
            pltpu.VMEM((ub,), jnp.int32),
            pltpu.VMEM((ub,), jnp.float32),
            pltpu.VMEM((ub,), jnp.float32),
            pltpu.VMEM((ub,), jnp.int32),
            pltpu.VMEM((ub,), jnp.int32),
            pltpu.VMEM((ub, TW), jnp.float32),
            pltpu.VMEM((ub, TW), jnp.float32),
            pltpu.VMEM((n_dblk, ubblk, n_dsub, n_bsub), jnp.float32),
            pltpu.VMEM((n_dblk, ubblk, n_dsub, n_bsub), jnp.float32),
            pltpu.SemaphoreType.DMA,
            pltpu.SemaphoreType.DMA,
            pltpu.SemaphoreType.DMA,
            pltpu.SemaphoreType.DMA,
            pltpu.SemaphoreType.DMA,
            pltpu.SemaphoreType.DMA,
        ],
    )
    def k(tbl_hbm, w_hbm, obs_hbm, feat_hbm, val_hbm, out_hbm,
          w_v, obs0, obs1, feat0, feat1, val0, val1, idx0, idx1,
          s0, s1, ot0, ot1, si0, si1, sg0, sg1, so0, so1):
        OBS, FEAT, VAL = [obs0, obs1], [feat0, feat1], [val0, val1]
        IDX, S, OT = [idx0, idx1], [s0, s1], [ot0, ot1]
        SI, SG, SO = [si0, si1], [sg0, sg1], [so0, so1]
        wid = lax.axis_index("s") * info.num_cores + lax.axis_index("c")
        pltpu.sync_copy(w_hbm, w_v)
        wregs = [w_v[pl.ds(t * LANES, LANES)] for t in range(d // LANES)]

        def unit_lq(u):
            gu = wid * per_w + u
            return gu // uq, gu % uq

        def issue_in(u, p):
            l, q = unit_lq(lax.min(u, per_w - 1))
            sl = pl.ds(q * ub, ub)
            pltpu.async_copy(obs_hbm.at[l, sl], OBS[p], SI[p])
            pltpu.async_copy(feat_hbm.at[l, sl], FEAT[p], SI[p])
            pltpu.async_copy(val_hbm.at[l, sl], VAL[p], SI[p])

        def wait_in(p):
            sl = pl.ds(0, ub)
            pltpu.make_async_copy(obs_hbm.at[0, sl], OBS[p], SI[p]).wait()
            pltpu.make_async_copy(feat_hbm.at[0, sl], FEAT[p], SI[p]).wait()
            pltpu.make_async_copy(val_hbm.at[0, sl], VAL[p], SI[p]).wait()

        def combine(p):
            for i in range(n_grp):
                sl = pl.ds(i * LANES, LANES)
                IDX[p][sl] = OBS[p][sl] * di + FEAT[p][sl]

        def issue_gather(p):
            for j in range(n_sub):
                sl = pl.ds(j * GATHER_SUB, GATHER_SUB)
                pltpu.async_copy(tbl_hbm.at[IDX[p].at[sl]], S[p].at[sl], SG[p])

        def wait_gather(p):
            for j in range(n_sub):
                sl = pl.ds(j * GATHER_SUB, GATHER_SUB)
                pltpu.make_async_copy(tbl_hbm.at[IDX[p].at[sl]], S[p].at[sl],
                                      SG[p]).wait()

        def issue_out(u, p):
            l, q = unit_lq(u)
            for dblk in range(n_dblk):
                pltpu.async_copy(OT[p].at[dblk],
                                 out_hbm.at[l, dblk, pl.ds(q * ubblk, ubblk)],
                                 SO[p])

        def wait_out(p):
            for dblk in range(n_dblk):
                pltpu.make_async_copy(
                    OT[p].at[dblk],
                    out_hbm.at[0, dblk, pl.ds(0, ubblk)], SO[p]).wait()

        def compute(p):
            s_v, ot = S[p], OT[p]

            @plsc.parallel_loop(0, n_grp)
            def grp_body(j):
                val16 = VAL[p][pl.ds(j * LANES, LANES)]
                riota = lax.iota(jnp.int32, LANES) + j * LANES
                jj = j // 8
                cc = (j % 8) * LANES
                for dg in range(d // LANES):
                    wreg = wregs[dg]
                    for kk in range(LANES):
                        d_i = dg * LANES + kk
                        wbc = lax.gather(
                            wreg, jnp.full((LANES, 1), kk, jnp.int32),
                            dimension_numbers=lax.GatherDimensionNumbers(
                                offset_dims=(), collapsed_slice_dims=(0,),
                                start_index_map=(0,)),
                            slice_sizes=(1,),
                            mode=lax.GatherScatterMode.PROMISE_IN_BOUNDS)
                        col = plsc.load_gather(
                            s_v, [riota, jnp.full((LANES,), d_i, jnp.int32)])
                        ot[d_i // 8, jj, d_i % 8, pl.ds(cc, LANES)] = (
                            col + val16 * wbc)

        issue_in(0, 0)
        issue_in(1, 1)
        wait_in(0)
        combine(0)
        issue_gather(0)
        wait_in(1)
        combine(1)
        issue_gather(1)
        wait_gather(0)
        compute(0)
        issue_out(0, 0)
        issue_in(2, 0)
        wait_in(0)
        combine(0)
        issue_gather(0)
        wait_gather(1)
        compute(1)
        issue_out(1, 1)
        issue_in(3, 1)

        def pair_body(i, _):
            u0 = 2 * i
            wait_in(1)
            combine(1)
            issue_gather(1)
            wait_gather(0)
            wait_out(0)
            compute(0)
            issue_out(u0, 0)
            issue_in(u0 + 2, 0)
            wait_in(0)
            combine(0)
            issue_gather(0)
            wait_gather(1)
            wait_out(1)
            compute(1)
            issue_out(u0 + 1, 1)
            issue_in(u0 + 3, 1)
            return 0

        lax.fori_loop(1, per_w // 2, pair_body, 0)

        wait_gather(0)
        wait_in(1)
        wait_out(0)
        wait_out(1)

    return k(tbl, w, obs_t, feat_t, val_t)


def kernel(val, obs_idx, feat_idx, W_val, b_val, emb_obs, emb_feat):
    b, l, _ = val.shape
    d = emb_obs.shape[1]
    di = emb_feat.shape[0]
    tbl = _build_table(emb_obs, emb_feat, b_val)
    out5 = _sc_embed5(
        tbl,
        W_val.reshape(d),
        obs_idx.astype(jnp.int32).T,
        feat_idx.astype(jnp.int32).T,
        val.reshape(b, l).T,
        di=di,
    )
    return out5.transpose(2, 4, 0, 1, 3).reshape(b, l, d)
